# baseline (speedup 1.0000x reference)
import functools

import jax
import jax.numpy as jnp
from jax import lax
from jax.experimental import pallas as pl
from jax.experimental.pallas import tpu as pltpu
from jax.experimental.pallas import tpu_sc as plsc

N = 10000
D = 128
E = 320000
NP = 10240
EP = 323584
CH = 2048
CPB = 4
PW = EP // 32

_f32 = jnp.float32
_mesh = plsc.VectorSubcoreMesh(core_axis_name="c", subcore_axis_name="s")
_ncl = pltpu.CompilerParams(needs_layout_passes=False)


def _worker_id():
    return lax.axis_index("c") * 16 + lax.axis_index("s")


@functools.partial(
    pl.kernel,
    out_type=jax.ShapeDtypeStruct((32, NP), _f32),
    mesh=_mesh,
    compiler_params=_ncl,
    scratch_types=[
        pltpu.VMEM((PW,), jnp.int32),
        pltpu.VMEM((NP,), _f32),
    ],
)
def _sc_deg(dst_hbm, out_hbm, didx_v, hist_v):
    wid = _worker_id()
    zero16 = jnp.zeros((16,), _f32)
    one16 = jnp.full((16,), 1.0, _f32)

    def zbody(i, _):
        hist_v[pl.ds(i * 16, 16)] = zero16
        return 0

    lax.fori_loop(0, NP // 16, zbody, 0)
    pltpu.sync_copy(dst_hbm.at[pl.ds(wid * PW, PW)], didx_v)

    def gbody(g, _):
        d16 = didx_v[pl.ds(g * 16, 16)]
        plsc.addupdate_scatter(hist_v, [d16], one16)
        return 0

    lax.fori_loop(0, PW // 16, gbody, 0)
    pltpu.sync_copy(hist_v, out_hbm.at[wid])


@functools.partial(
    pl.kernel,
    out_type=jax.ShapeDtypeStruct((32, NP), _f32),
    mesh=_mesh,
    compiler_params=_ncl,
    scratch_types=[
        pltpu.VMEM((PW,), jnp.int32),
        pltpu.VMEM((PW,), _f32),
        pltpu.VMEM((NP,), _f32),
    ],
)
def _sc_denom(ex_hbm, dst_hbm, out_hbm, didx_v, exs_v, hist_v):
    wid = _worker_id()
    zero16 = jnp.zeros((16,), _f32)

    def zbody(i, _):
        hist_v[pl.ds(i * 16, 16)] = zero16
        return 0

    lax.fori_loop(0, NP // 16, zbody, 0)
    pltpu.sync_copy(dst_hbm.at[pl.ds(wid * PW, PW)], didx_v)
    pltpu.sync_copy(ex_hbm.at[pl.ds(wid * PW, PW)], exs_v)

    def gbody(g, _):
        d16 = didx_v[pl.ds(g * 16, 16)]
        ex16 = exs_v[pl.ds(g * 16, 16)]
        plsc.addupdate_scatter(hist_v, [d16], ex16)
        return 0

    lax.fori_loop(0, PW // 16, gbody, 0)
    pltpu.sync_copy(hist_v, out_hbm.at[wid])


@functools.partial(
    pl.kernel,
    out_type=jax.ShapeDtypeStruct((D, NP), _f32),
    mesh=_mesh,
    compiler_params=_ncl,
    scratch_types=[
        pltpu.VMEM((CH,), jnp.int32),
        pltpu.VMEM((CH,), jnp.int32),
        pltpu.VMEM((CPB, NP), _f32),
        pltpu.VMEM((CPB, NP), _f32),
        pltpu.SemaphoreType.DMA,
    ],
)
def _sc_segsum(gT_hbm, src_hbm, dst_hbm, out_hbm, sidx_v, didx_v, tab_v,
               acc_v, sem):
    wid = _worker_id()
    zero16 = jnp.zeros((16,), _f32)
    pltpu.async_copy(gT_hbm.at[pl.ds(wid * CPB, CPB)], acc_v, sem).wait()

    def lbody(i, _):
        sl = pl.ds(i * 16, 16)
        for k in range(CPB):
            tab_v[k, sl] = acc_v[k, sl]
            acc_v[k, sl] = zero16
        return 0

    lax.fori_loop(0, NP // 16, lbody, 0)
    kidx = [jnp.full((16,), k, jnp.int32) for k in range(CPB)]

    def obody(o, _):
        base = o * CH
        pltpu.sync_copy(src_hbm.at[pl.ds(base, CH)], sidx_v)
        pltpu.sync_copy(dst_hbm.at[pl.ds(base, CH)], didx_v)

        @plsc.parallel_loop(0, CH // 16, unroll=8)
        def gbody(g):
            s16 = sidx_v[pl.ds(g * 16, 16)]
            d16 = didx_v[pl.ds(g * 16, 16)]
            for k in range(CPB):
                v = plsc.load_gather(tab_v, [kidx[k], s16])
                plsc.addupdate_scatter(acc_v, [kidx[k], d16], v)
        return 0

    lax.fori_loop(0, EP // CH, obody, 0)
    pltpu.sync_copy(acc_v, out_hbm.at[pl.ds(wid * CPB, CPB)])


@functools.partial(
    pl.kernel,
    out_type=jax.ShapeDtypeStruct((D, NP), _f32),
    mesh=_mesh,
    compiler_params=_ncl,
    scratch_types=[
        pltpu.VMEM((CH,), jnp.int32),
        pltpu.VMEM((CH,), jnp.int32),
        pltpu.VMEM((CH,), _f32),
        pltpu.VMEM((CPB, NP), _f32),
        pltpu.VMEM((CPB, NP), _f32),
        pltpu.SemaphoreType.DMA,
    ],
)
def _sc_aggregate(fsT_hbm, ex_hbm, src_hbm, dst_hbm, out_hbm, sidx_v, didx_v,
                  exs_v, tab_v, acc_v, sem):
    wid = _worker_id()
    zero16 = jnp.zeros((16,), _f32)
    pltpu.async_copy(fsT_hbm.at[pl.ds(wid * CPB, CPB)], acc_v, sem).wait()

    def lbody(i, _):
        sl = pl.ds(i * 16, 16)
        for k in range(CPB):
            tab_v[k, sl] = acc_v[k, sl]
            acc_v[k, sl] = zero16
        return 0

    lax.fori_loop(0, NP // 16, lbody, 0)
    kidx = [jnp.full((16,), k, jnp.int32) for k in range(CPB)]

    def obody(o, _):
        base = o * CH
        pltpu.sync_copy(src_hbm.at[pl.ds(base, CH)], sidx_v)
        pltpu.sync_copy(dst_hbm.at[pl.ds(base, CH)], didx_v)
        pltpu.sync_copy(ex_hbm.at[pl.ds(base, CH)], exs_v)

        @plsc.parallel_loop(0, CH // 16, unroll=8)
        def gbody(g):
            s16 = sidx_v[pl.ds(g * 16, 16)]
            d16 = didx_v[pl.ds(g * 16, 16)]
            ex16 = exs_v[pl.ds(g * 16, 16)]
            for k in range(CPB):
                v = plsc.load_gather(tab_v, [kidx[k], s16]) * ex16
                plsc.addupdate_scatter(acc_v, [kidx[k], d16], v)
        return 0

    lax.fori_loop(0, EP // CH, obody, 0)
    pltpu.sync_copy(acc_v, out_hbm.at[pl.ds(wid * CPB, CPB)])


ECH = 128
NCHUNK = PW // ECH


@functools.partial(
    pl.kernel,
    out_type=jax.ShapeDtypeStruct((EP, D), _f32),
    mesh=_mesh,
    scratch_types=[
        pltpu.VMEM((ECH,), jnp.int32),
        pltpu.VMEM((ECH,), jnp.int32),
        pltpu.VMEM((ECH, D), _f32),
        pltpu.VMEM((ECH, D), _f32),
        pltpu.VMEM((D,), _f32),
        pltpu.SemaphoreType.DMA,
        pltpu.SemaphoreType.DMA,
    ],
)
def _sc_edge_w(fs_hbm, fd_hbm, src_hbm, dst_hbm, attn_hbm, w_hbm,
               sidx_v, didx_v, fsr_v, fdr_v, attn_v, sem1, sem2):
    wid = _worker_id()
    pltpu.sync_copy(attn_hbm, attn_v)
    a1 = [attn_v[pl.ds(t * 16, 16)] * 0.6 for t in range(D // 16)]
    a2 = [attn_v[pl.ds(t * 16, 16)] * 0.4 for t in range(D // 16)]

    def body(i, _):
        base = wid * PW + i * ECH
        pltpu.sync_copy(src_hbm.at[pl.ds(base, ECH)], sidx_v)
        pltpu.sync_copy(dst_hbm.at[pl.ds(base, ECH)], didx_v)
        cp1 = pltpu.async_copy(fs_hbm.at[sidx_v], fsr_v, sem1)
        cp2 = pltpu.async_copy(fd_hbm.at[didx_v], fdr_v, sem2)
        cp1.wait()
        cp2.wait()

        @plsc.parallel_loop(0, ECH, unroll=2)
        def ebody(j):
            for t in range(D // 16):
                sl = pl.ds(t * 16, 16)
                zz = fsr_v[j, sl] + fdr_v[j, sl]
                fsr_v[j, sl] = zz * a1[t] + jnp.abs(zz) * a2[t]
        pltpu.sync_copy(fsr_v, w_hbm.at[pl.ds(base, ECH)])
        return 0

    lax.fori_loop(0, NCHUNK, body, 0)


_R = 512
_GRID = NP // _R


def _dis_of(deg_ref):
    deg = jnp.sum(deg_ref[...], axis=0)[:, None]
    return lax.rsqrt(jnp.maximum(deg, 1.0))


def _leaky(x, slope):
    return jnp.where(x >= 0, x, slope * x)


def _tc_scale_body(deg_ref, f_ref, o_ref):
    o_ref[...] = (f_ref[...] * _dis_of(deg_ref)).T


def _tc_x1_body(rec_ref, s_ref, f_ref, deg_ref, x1_ref, g_ref):
    re = rec_ref[0, 0]
    dis = _dis_of(deg_ref)
    htil = s_ref[...].T * dis
    x1 = htil * (-re) + f_ref[...] * (re - 1.0)
    x1_ref[...] = x1
    g_ref[...] = (x1 * dis).T


def _cheb_h(rec_ref, s_ref, x1_ref, f_ref, deg_ref, wc_ref, bc_ref):
    re = rec_ref[0, 0]
    dis = _dis_of(deg_ref)
    htil = s_ref[...].T * dis
    x1 = x1_ref[...]
    f = f_ref[...]
    x2 = htil * (-2.0 * re) + x1 * (2.0 * re - 1.0) - f
    xt = jnp.concatenate([f, x1, x2], axis=1)
    hh = jnp.dot(xt, wc_ref[...], preferred_element_type=_f32) + bc_ref[...]
    return _leaky(hh, 0.01), dis


def _tc_cheb_body(rec_ref, s_ref, x1_ref, f_ref, deg_ref, wc_ref, bc_ref,
                  h_ref, g_ref):
    h, dis = _cheb_h(rec_ref, s_ref, x1_ref, f_ref, deg_ref, wc_ref, bc_ref)
    h_ref[...] = h
    g_ref[...] = (h * dis).T


def _tc_gat_body(rec_ref, s_ref, x1_ref, f_ref, deg_ref, wc_ref, bc_ref,
                 ws_ref, bs_ref, wd_ref, bd_ref, fs_ref, fd_ref, fst_ref):
    h, _ = _cheb_h(rec_ref, s_ref, x1_ref, f_ref, deg_ref, wc_ref, bc_ref)
    fs = jnp.dot(h, ws_ref[...], preferred_element_type=_f32) + bs_ref[...]
    fs_ref[...] = fs
    fd_ref[...] = jnp.dot(h, wd_ref[...], preferred_element_type=_f32) + bd_ref[...]
    fst_ref[...] = fs.T


def _tc_exb_body(w_ref, o_ref):
    e = jnp.sum(w_ref[...], axis=1)
    o_ref[...] = jnp.exp(e).reshape(o_ref.shape)


def _tc_out_body(ft_ref, dn_ref, o_ref):
    v = ft_ref[...].T
    den = jnp.sum(dn_ref[...], axis=0)[:, None]
    invd = jnp.where(den > 0.0, 1.0 / den, 0.0)
    o_ref[...] = _leaky(v * invd, 0.01)


def _bs_rows(width):
    return pl.BlockSpec((_R, width), lambda i: (i, 0))


_bs_T = pl.BlockSpec((D, _R), lambda i: (0, i))
_bs_deg = pl.BlockSpec((32, _R), lambda i: (0, i))
_bs_smem = pl.BlockSpec(memory_space=pltpu.MemorySpace.SMEM)


def _bs_full(shape):
    nd = len(shape)
    return pl.BlockSpec(shape, lambda i, _nd=nd: (0,) * nd)


def _tc_scale(deg, f):
    return pl.pallas_call(
        _tc_scale_body,
        grid=(_GRID,),
        in_specs=[_bs_deg, _bs_rows(D)],
        out_specs=_bs_T,
        out_shape=jax.ShapeDtypeStruct((D, NP), _f32),
    )(deg, f)


def _tc_x1(rec, sT, f, deg):
    return pl.pallas_call(
        _tc_x1_body,
        grid=(_GRID,),
        in_specs=[_bs_smem, _bs_T, _bs_rows(D), _bs_deg],
        out_specs=[_bs_rows(D), _bs_T],
        out_shape=[jax.ShapeDtypeStruct((NP, D), _f32),
                   jax.ShapeDtypeStruct((D, NP), _f32)],
    )(rec, sT, f, deg)


def _tc_cheb(rec, sT, x1, f, deg, wc, bc):
    return pl.pallas_call(
        _tc_cheb_body,
        grid=(_GRID,),
        in_specs=[_bs_smem, _bs_T, _bs_rows(D), _bs_rows(D), _bs_deg,
                  _bs_full((3 * D, D)), _bs_full((1, D))],
        out_specs=[_bs_rows(D), _bs_T],
        out_shape=[jax.ShapeDtypeStruct((NP, D), _f32),
                   jax.ShapeDtypeStruct((D, NP), _f32)],
    )(rec, sT, x1, f, deg, wc, bc)


def _tc_gat(rec, sT, x1, f, deg, wc, bc, ws, bs, wd, bd):
    return pl.pallas_call(
        _tc_gat_body,
        grid=(_GRID,),
        in_specs=[_bs_smem, _bs_T, _bs_rows(D), _bs_rows(D), _bs_deg,
                  _bs_full((3 * D, D)), _bs_full((1, D)),
                  _bs_full((D, D)), _bs_full((1, D)),
                  _bs_full((D, D)), _bs_full((1, D))],
        out_specs=[_bs_rows(D), _bs_rows(D), _bs_T],
        out_shape=[jax.ShapeDtypeStruct((NP, D), _f32),
                   jax.ShapeDtypeStruct((NP, D), _f32),
                   jax.ShapeDtypeStruct((D, NP), _f32)],
    )(rec, sT, x1, f, deg, wc, bc, ws, bs, wd, bd)


_RE = 2048


def _tc_exb(w):
    return pl.pallas_call(
        _tc_exb_body,
        grid=(EP // _RE,),
        in_specs=[pl.BlockSpec((_RE, D), lambda i: (i, 0))],
        out_specs=pl.BlockSpec((_RE // 128, 128), lambda i: (i, 0)),
        out_shape=jax.ShapeDtypeStruct((EP // 128, 128), _f32),
    )(w)


def _tc_out(ftT, dn):
    return pl.pallas_call(
        _tc_out_body,
        grid=(_GRID,),
        in_specs=[_bs_T, _bs_deg],
        out_specs=_bs_rows(D),
        out_shape=jax.ShapeDtypeStruct((NP, D), _f32),
    )(ftT, dn)


def kernel(embedding, laplacian_lambda_max, edge_index, W_cheb, b_cheb,
           W_src, b_src, W_dst, b_dst, attn):
    src = edge_index[0]
    dst = edge_index[1]
    pad = jnp.full((EP - E,), N, jnp.int32)
    srcp = jnp.concatenate([src, pad])
    dstp = jnp.concatenate([dst, pad])
    embp = jnp.pad(embedding, ((0, NP - N), (0, 0)))
    rec = jnp.reshape(2.0 / laplacian_lambda_max[0], (1, 1))
    bc = jnp.reshape(b_cheb, (1, D))
    bs = jnp.reshape(b_src, (1, D))
    bd = jnp.reshape(b_dst, (1, D))
    attn_flat = jnp.reshape(attn, (D,))

    deg = _sc_deg(dstp)
    g0T = _tc_scale(deg, embp)
    s0T = _sc_segsum(g0T, srcp, dstp)
    x1, g1T = _tc_x1(rec, s0T, embp, deg)
    s1T = _sc_segsum(g1T, srcp, dstp)
    h1, g0bT = _tc_cheb(rec, s1T, x1, embp, deg, W_cheb, bc)
    s0bT = _sc_segsum(g0bT, srcp, dstp)
    x1b, g1bT = _tc_x1(rec, s0bT, h1, deg)
    s1bT = _sc_segsum(g1bT, srcp, dstp)
    fs, fd, fsT = _tc_gat(rec, s1bT, x1b, h1, deg, W_cheb, bc,
                          W_src, bs, W_dst, bd)
    w = _sc_edge_w(fs, fd, srcp, dstp, attn_flat)
    ex = jnp.reshape(_tc_exb(w), (EP,))
    ftT = _sc_aggregate(fsT, ex, srcp, dstp)
    dn = _sc_denom(ex, dstp)
    rst = _tc_out(ftT, dn)
    return rst[:N]


# speedup vs baseline: 1.2061x; 1.2061x over previous
import functools

import jax
import jax.numpy as jnp
from jax import lax
from jax.experimental import pallas as pl
from jax.experimental.pallas import tpu as pltpu
from jax.experimental.pallas import tpu_sc as plsc

N = 10000
D = 128
E = 320000
NP = 10240
EP = 323584
CH = 4096
CPB = 4
PW = EP // 32

_f32 = jnp.float32
_mesh = plsc.VectorSubcoreMesh(core_axis_name="c", subcore_axis_name="s")
_ncl = pltpu.CompilerParams(needs_layout_passes=False)


def _worker_id():
    return lax.axis_index("c") * 16 + lax.axis_index("s")


@functools.partial(
    pl.kernel,
    out_type=jax.ShapeDtypeStruct((32, NP), _f32),
    mesh=_mesh,
    compiler_params=_ncl,
    scratch_types=[
        pltpu.VMEM((PW,), jnp.int32),
        pltpu.VMEM((NP,), _f32),
    ],
)
def _sc_deg(dst_hbm, out_hbm, didx_v, hist_v):
    wid = _worker_id()
    zero16 = jnp.zeros((16,), _f32)
    one16 = jnp.full((16,), 1.0, _f32)

    def zbody(i, _):
        hist_v[pl.ds(i * 16, 16)] = zero16
        return 0

    lax.fori_loop(0, NP // 16, zbody, 0)
    pltpu.sync_copy(dst_hbm.at[pl.ds(wid * PW, PW)], didx_v)

    def gbody(g, _):
        d16 = didx_v[pl.ds(g * 16, 16)]
        plsc.addupdate_scatter(hist_v, [d16], one16)
        return 0

    lax.fori_loop(0, PW // 16, gbody, 0)
    pltpu.sync_copy(hist_v, out_hbm.at[wid])


@functools.partial(
    pl.kernel,
    out_type=jax.ShapeDtypeStruct((32, NP), _f32),
    mesh=_mesh,
    compiler_params=_ncl,
    scratch_types=[
        pltpu.VMEM((PW,), jnp.int32),
        pltpu.VMEM((PW,), _f32),
        pltpu.VMEM((NP,), _f32),
    ],
)
def _sc_denom(ex_hbm, dst_hbm, out_hbm, didx_v, exs_v, hist_v):
    wid = _worker_id()
    zero16 = jnp.zeros((16,), _f32)

    def zbody(i, _):
        hist_v[pl.ds(i * 16, 16)] = zero16
        return 0

    lax.fori_loop(0, NP // 16, zbody, 0)
    pltpu.sync_copy(dst_hbm.at[pl.ds(wid * PW, PW)], didx_v)
    pltpu.sync_copy(ex_hbm.at[pl.ds(wid * PW, PW)], exs_v)

    def gbody(g, _):
        d16 = didx_v[pl.ds(g * 16, 16)]
        ex16 = exs_v[pl.ds(g * 16, 16)]
        plsc.addupdate_scatter(hist_v, [d16], ex16)
        return 0

    lax.fori_loop(0, PW // 16, gbody, 0)
    pltpu.sync_copy(hist_v, out_hbm.at[wid])


@functools.partial(
    pl.kernel,
    out_type=jax.ShapeDtypeStruct((D, NP), _f32),
    mesh=_mesh,
    compiler_params=_ncl,
    scratch_types=[
        pltpu.VMEM((CH,), jnp.int32),
        pltpu.VMEM((CH,), jnp.int32),
        pltpu.VMEM((CPB, NP), _f32),
        pltpu.VMEM((CPB, NP), _f32),
        pltpu.SemaphoreType.DMA,
    ],
)
def _sc_segsum(gT_hbm, src_hbm, dst_hbm, out_hbm, sidx_v, didx_v, tab_v,
               acc_v, sem):
    wid = _worker_id()
    zero16 = jnp.zeros((16,), _f32)
    pltpu.async_copy(gT_hbm.at[pl.ds(wid * CPB, CPB)], acc_v, sem).wait()

    def lbody(i, _):
        sl = pl.ds(i * 16, 16)
        for k in range(CPB):
            tab_v[k, sl] = acc_v[k, sl]
            acc_v[k, sl] = zero16
        return 0

    lax.fori_loop(0, NP // 16, lbody, 0)
    kidx = [jnp.full((16,), k, jnp.int32) for k in range(CPB)]

    def obody(o, _):
        base = o * CH
        pltpu.sync_copy(src_hbm.at[pl.ds(base, CH)], sidx_v)
        pltpu.sync_copy(dst_hbm.at[pl.ds(base, CH)], didx_v)

        @plsc.parallel_loop(0, CH // 16, unroll=8)
        def gbody(g):
            s16 = sidx_v[pl.ds(g * 16, 16)]
            d16 = didx_v[pl.ds(g * 16, 16)]
            for k in range(CPB):
                v = plsc.load_gather(tab_v, [kidx[k], s16])
                plsc.addupdate_scatter(acc_v, [kidx[k], d16], v)
        return 0

    lax.fori_loop(0, EP // CH, obody, 0)
    pltpu.sync_copy(acc_v, out_hbm.at[pl.ds(wid * CPB, CPB)])


@functools.partial(
    pl.kernel,
    out_type=jax.ShapeDtypeStruct((D, NP), _f32),
    mesh=_mesh,
    compiler_params=_ncl,
    scratch_types=[
        pltpu.VMEM((CH,), jnp.int32),
        pltpu.VMEM((CH,), jnp.int32),
        pltpu.VMEM((CH,), _f32),
        pltpu.VMEM((CPB, NP), _f32),
        pltpu.VMEM((CPB, NP), _f32),
        pltpu.SemaphoreType.DMA,
    ],
)
def _sc_aggregate(fsT_hbm, ex_hbm, src_hbm, dst_hbm, out_hbm, sidx_v, didx_v,
                  exs_v, tab_v, acc_v, sem):
    wid = _worker_id()
    zero16 = jnp.zeros((16,), _f32)
    pltpu.async_copy(fsT_hbm.at[pl.ds(wid * CPB, CPB)], acc_v, sem).wait()

    def lbody(i, _):
        sl = pl.ds(i * 16, 16)
        for k in range(CPB):
            tab_v[k, sl] = acc_v[k, sl]
            acc_v[k, sl] = zero16
        return 0

    lax.fori_loop(0, NP // 16, lbody, 0)
    kidx = [jnp.full((16,), k, jnp.int32) for k in range(CPB)]

    def obody(o, _):
        base = o * CH
        pltpu.sync_copy(src_hbm.at[pl.ds(base, CH)], sidx_v)
        pltpu.sync_copy(dst_hbm.at[pl.ds(base, CH)], didx_v)
        pltpu.sync_copy(ex_hbm.at[pl.ds(base, CH)], exs_v)

        @plsc.parallel_loop(0, CH // 16, unroll=8)
        def gbody(g):
            s16 = sidx_v[pl.ds(g * 16, 16)]
            d16 = didx_v[pl.ds(g * 16, 16)]
            ex16 = exs_v[pl.ds(g * 16, 16)]
            for k in range(CPB):
                v = plsc.load_gather(tab_v, [kidx[k], s16]) * ex16
                plsc.addupdate_scatter(acc_v, [kidx[k], d16], v)
        return 0

    lax.fori_loop(0, EP // CH, obody, 0)
    pltpu.sync_copy(acc_v, out_hbm.at[pl.ds(wid * CPB, CPB)])


ECH = 128
EOUT = 256


@functools.partial(
    pl.kernel,
    out_type=jax.ShapeDtypeStruct((EP, D), _f32),
    mesh=_mesh,
    scratch_types=[
        pltpu.VMEM((EOUT,), jnp.int32),
        pltpu.VMEM((EOUT,), jnp.int32),
        pltpu.VMEM((EOUT, D), _f32),
        pltpu.VMEM((EOUT, D), _f32),
        pltpu.VMEM((D,), _f32),
        pltpu.SemaphoreType.DMA,
        pltpu.SemaphoreType.DMA,
    ],
)
def _sc_edge_w(fs_hbm, fd_hbm, src_hbm, dst_hbm, attn_hbm, w_hbm,
               sidx_v, didx_v, fsr_v, fdr_v, attn_v, sem1, sem2):
    wid = _worker_id()
    pltpu.sync_copy(attn_hbm, attn_v)
    a1 = [attn_v[pl.ds(t * 16, 16)] * 0.6 for t in range(D // 16)]
    a2 = [attn_v[pl.ds(t * 16, 16)] * 0.4 for t in range(D // 16)]

    def body(i, _):
        base = wid * PW + i * EOUT
        pltpu.sync_copy(src_hbm.at[pl.ds(base, EOUT)], sidx_v)
        pltpu.sync_copy(dst_hbm.at[pl.ds(base, EOUT)], didx_v)
        cps = []
        for q in range(EOUT // ECH):
            sl = pl.ds(q * ECH, ECH)
            cps.append(pltpu.async_copy(fs_hbm.at[sidx_v.at[sl]],
                                        fsr_v.at[sl], sem1))
            cps.append(pltpu.async_copy(fd_hbm.at[didx_v.at[sl]],
                                        fdr_v.at[sl], sem2))
        for cp in cps:
            cp.wait()

        @plsc.parallel_loop(0, EOUT, unroll=2)
        def ebody(j):
            for t in range(D // 16):
                sl = pl.ds(t * 16, 16)
                zz = fsr_v[j, sl] + fdr_v[j, sl]
                fsr_v[j, sl] = zz * a1[t] + jnp.abs(zz) * a2[t]
        pltpu.sync_copy(fsr_v, w_hbm.at[pl.ds(base, EOUT)])
        return 0

    lax.fori_loop(0, PW // EOUT, body, 0)


_R = 512
_GRID = NP // _R


def _dis_of(deg_ref):
    deg = jnp.sum(deg_ref[...], axis=0)[:, None]
    return lax.rsqrt(jnp.maximum(deg, 1.0))


def _leaky(x, slope):
    return jnp.where(x >= 0, x, slope * x)


def _tc_scale_body(deg_ref, f_ref, o_ref):
    o_ref[...] = (f_ref[...] * _dis_of(deg_ref)).T


def _tc_x1_body(rec_ref, s_ref, f_ref, deg_ref, x1_ref, g_ref):
    re = rec_ref[0, 0]
    dis = _dis_of(deg_ref)
    htil = s_ref[...].T * dis
    x1 = htil * (-re) + f_ref[...] * (re - 1.0)
    x1_ref[...] = x1
    g_ref[...] = (x1 * dis).T


def _cheb_h(rec_ref, s_ref, x1_ref, f_ref, deg_ref, wc_ref, bc_ref):
    re = rec_ref[0, 0]
    dis = _dis_of(deg_ref)
    htil = s_ref[...].T * dis
    x1 = x1_ref[...]
    f = f_ref[...]
    x2 = htil * (-2.0 * re) + x1 * (2.0 * re - 1.0) - f
    xt = jnp.concatenate([f, x1, x2], axis=1)
    hh = jnp.dot(xt, wc_ref[...], preferred_element_type=_f32) + bc_ref[...]
    return _leaky(hh, 0.01), dis


def _tc_cheb_body(rec_ref, s_ref, x1_ref, f_ref, deg_ref, wc_ref, bc_ref,
                  h_ref, g_ref):
    h, dis = _cheb_h(rec_ref, s_ref, x1_ref, f_ref, deg_ref, wc_ref, bc_ref)
    h_ref[...] = h
    g_ref[...] = (h * dis).T


def _tc_gat_body(rec_ref, s_ref, x1_ref, f_ref, deg_ref, wc_ref, bc_ref,
                 ws_ref, bs_ref, wd_ref, bd_ref, fs_ref, fd_ref, fst_ref):
    h, _ = _cheb_h(rec_ref, s_ref, x1_ref, f_ref, deg_ref, wc_ref, bc_ref)
    fs = jnp.dot(h, ws_ref[...], preferred_element_type=_f32) + bs_ref[...]
    fs_ref[...] = fs
    fd_ref[...] = jnp.dot(h, wd_ref[...], preferred_element_type=_f32) + bd_ref[...]
    fst_ref[...] = fs.T


def _tc_exb_body(w_ref, o_ref):
    e = jnp.sum(w_ref[...], axis=1)
    o_ref[...] = jnp.exp(e).reshape(o_ref.shape)


def _tc_out_body(ft_ref, dn_ref, o_ref):
    v = ft_ref[...].T
    den = jnp.sum(dn_ref[...], axis=0)[:, None]
    invd = jnp.where(den > 0.0, 1.0 / den, 0.0)
    o_ref[...] = _leaky(v * invd, 0.01)


def _bs_rows(width):
    return pl.BlockSpec((_R, width), lambda i: (i, 0))


_bs_T = pl.BlockSpec((D, _R), lambda i: (0, i))
_bs_deg = pl.BlockSpec((32, _R), lambda i: (0, i))
_bs_smem = pl.BlockSpec(memory_space=pltpu.MemorySpace.SMEM)


def _bs_full(shape):
    nd = len(shape)
    return pl.BlockSpec(shape, lambda i, _nd=nd: (0,) * nd)


def _tc_scale(deg, f):
    return pl.pallas_call(
        _tc_scale_body,
        grid=(_GRID,),
        in_specs=[_bs_deg, _bs_rows(D)],
        out_specs=_bs_T,
        out_shape=jax.ShapeDtypeStruct((D, NP), _f32),
    )(deg, f)


def _tc_x1(rec, sT, f, deg):
    return pl.pallas_call(
        _tc_x1_body,
        grid=(_GRID,),
        in_specs=[_bs_smem, _bs_T, _bs_rows(D), _bs_deg],
        out_specs=[_bs_rows(D), _bs_T],
        out_shape=[jax.ShapeDtypeStruct((NP, D), _f32),
                   jax.ShapeDtypeStruct((D, NP), _f32)],
    )(rec, sT, f, deg)


def _tc_cheb(rec, sT, x1, f, deg, wc, bc):
    return pl.pallas_call(
        _tc_cheb_body,
        grid=(_GRID,),
        in_specs=[_bs_smem, _bs_T, _bs_rows(D), _bs_rows(D), _bs_deg,
                  _bs_full((3 * D, D)), _bs_full((1, D))],
        out_specs=[_bs_rows(D), _bs_T],
        out_shape=[jax.ShapeDtypeStruct((NP, D), _f32),
                   jax.ShapeDtypeStruct((D, NP), _f32)],
    )(rec, sT, x1, f, deg, wc, bc)


def _tc_gat(rec, sT, x1, f, deg, wc, bc, ws, bs, wd, bd):
    return pl.pallas_call(
        _tc_gat_body,
        grid=(_GRID,),
        in_specs=[_bs_smem, _bs_T, _bs_rows(D), _bs_rows(D), _bs_deg,
                  _bs_full((3 * D, D)), _bs_full((1, D)),
                  _bs_full((D, D)), _bs_full((1, D)),
                  _bs_full((D, D)), _bs_full((1, D))],
        out_specs=[_bs_rows(D), _bs_rows(D), _bs_T],
        out_shape=[jax.ShapeDtypeStruct((NP, D), _f32),
                   jax.ShapeDtypeStruct((NP, D), _f32),
                   jax.ShapeDtypeStruct((D, NP), _f32)],
    )(rec, sT, x1, f, deg, wc, bc, ws, bs, wd, bd)


_RE = 2048


def _tc_exb(w):
    return pl.pallas_call(
        _tc_exb_body,
        grid=(EP // _RE,),
        in_specs=[pl.BlockSpec((_RE, D), lambda i: (i, 0))],
        out_specs=pl.BlockSpec((_RE // 128, 128), lambda i: (i, 0)),
        out_shape=jax.ShapeDtypeStruct((EP // 128, 128), _f32),
    )(w)


def _tc_out(ftT, dn):
    return pl.pallas_call(
        _tc_out_body,
        grid=(_GRID,),
        in_specs=[_bs_T, _bs_deg],
        out_specs=_bs_rows(D),
        out_shape=jax.ShapeDtypeStruct((NP, D), _f32),
    )(ftT, dn)


def kernel(embedding, laplacian_lambda_max, edge_index, W_cheb, b_cheb,
           W_src, b_src, W_dst, b_dst, attn):
    src = edge_index[0]
    dst = edge_index[1]
    pad = jnp.full((EP - E,), N, jnp.int32)
    srcp = jnp.concatenate([src, pad])
    dstp = jnp.concatenate([dst, pad])
    embp = jnp.pad(embedding, ((0, NP - N), (0, 0)))
    rec = jnp.reshape(2.0 / laplacian_lambda_max[0], (1, 1))
    bc = jnp.reshape(b_cheb, (1, D))
    bs = jnp.reshape(b_src, (1, D))
    bd = jnp.reshape(b_dst, (1, D))
    attn_flat = jnp.reshape(attn, (D,))

    deg = _sc_deg(dstp)
    g0T = _tc_scale(deg, embp)
    s0T = _sc_segsum(g0T, srcp, dstp)
    x1, g1T = _tc_x1(rec, s0T, embp, deg)
    s1T = _sc_segsum(g1T, srcp, dstp)
    h1, g0bT = _tc_cheb(rec, s1T, x1, embp, deg, W_cheb, bc)
    s0bT = _sc_segsum(g0bT, srcp, dstp)
    x1b, g1bT = _tc_x1(rec, s0bT, h1, deg)
    s1bT = _sc_segsum(g1bT, srcp, dstp)
    fs, fd, fsT = _tc_gat(rec, s1bT, x1b, h1, deg, W_cheb, bc,
                          W_src, bs, W_dst, bd)
    w = _sc_edge_w(fs, fd, srcp, dstp, attn_flat)
    ex = jnp.reshape(_tc_exb(w), (EP,))
    ftT = _sc_aggregate(fsT, ex, srcp, dstp)
    dn = _sc_denom(ex, dstp)
    rst = _tc_out(ftT, dn)
    return rst[:N]
